# baseline (device time: 14704 ns/iter reference)
import jax
import jax.numpy as jnp
from jax import lax
from jax.experimental import pallas as pl
from jax.experimental.pallas import tpu as pltpu

B, SQ, SKV, H, D = 8, 1, 512, 8, 64
NB = B // 2


def kernel(Q, K, V):
    scale = D ** -0.5
    Kt = jnp.transpose(K, (0, 2, 3, 1))
    Vt = jnp.transpose(V, (0, 2, 3, 1))

    def body(q_ref, k_ref, v_ref, out_ref, kbuf, vbuf, qbuf,
             send_buf, ypart, xpart, dpart, obuf,
             sems, ysend_sems, yrecv_sems, xsend_sems, xrecv_sems,
             dsend_sems, drecv_sems, out_sems, q_sem):
        my_x = lax.axis_index("x")
        my_y = lax.axis_index("y")
        my_z = lax.axis_index("z")
        nbr_y = (my_x, 1 - my_y, my_z)
        nbr_x = (1 - my_x, my_y, my_z)
        nbr_d = (1 - my_x, 1 - my_y, my_z)
        b0 = my_x * NB

        barrier_sem = pltpu.get_barrier_semaphore()
        for nbr in (nbr_y, nbr_x, nbr_d):
            pl.semaphore_signal(
                barrier_sem, inc=1, device_id=nbr,
                device_id_type=pl.DeviceIdType.MESH,
            )

        qc = pltpu.make_async_copy(q_ref, qbuf, q_sem)
        qc.start()
        copies = []
        for i in range(NB):
            kc = pltpu.make_async_copy(
                k_ref.at[b0 + i], kbuf.at[i], sems.at[i]
            )
            vc = pltpu.make_async_copy(
                v_ref.at[b0 + i], vbuf.at[i], sems.at[NB + i]
            )
            kc.start()
            vc.start()
            copies.append((kc, vc))

        rdmas = []
        for i in range(NB):
            kc, vc = copies[i]
            if i == 0:
                qc.wait()
            q_b = qbuf[pl.ds(b0 + i, 1)].reshape(H, D)
            kc.wait()
            s = lax.dot_general(
                q_b, kbuf[i],
                (((1,), (1,)), ((0,), (0,))),
                preferred_element_type=jnp.float32,
            ) * scale
            m = jnp.max(s, axis=-1, keepdims=True)
            p = jnp.exp(s - m)
            l = jnp.sum(p, axis=-1, keepdims=True)
            vc.wait()
            o = lax.dot_general(
                p, vbuf[i],
                (((1,), (2,)), ((0,), (0,))),
                preferred_element_type=jnp.float32,
            )
            send_buf[i, :, 0:D] = o
            send_buf[i, :, D:2 * D] = jnp.broadcast_to(m, (H, D))
            send_buf[i, :, 2 * D:3 * D] = jnp.broadcast_to(l, (H, D))
            if i == 0:
                pl.semaphore_wait(barrier_sem, 3)
            for dst, ss, rs, nbr in (
                (ypart, ysend_sems, yrecv_sems, nbr_y),
                (xpart, xsend_sems, xrecv_sems, nbr_x),
                (dpart, dsend_sems, drecv_sems, nbr_d),
            ):
                rdma = pltpu.make_async_remote_copy(
                    src_ref=send_buf.at[i],
                    dst_ref=dst.at[i],
                    send_sem=ss.at[i],
                    recv_sem=rs.at[i],
                    device_id=nbr,
                    device_id_type=pl.DeviceIdType.MESH,
                )
                rdma.start()
                rdmas.append(rdma)

        def combine(mine, other):
            o1, m1, l1 = (mine[:, 0:D], mine[:, D:2 * D], mine[:, 2 * D:3 * D])
            o2, m2, l2 = (other[:, 0:D], other[:, D:2 * D], other[:, 2 * D:3 * D])
            mn = jnp.maximum(m1, m2)
            a1 = jnp.exp(m1 - mn)
            a2 = jnp.exp(m2 - mn)
            return (a1 * o1 + a2 * o2) / (a1 * l1 + a2 * l2)

        out_copies = []
        for i in range(NB):
            rdmas[3 * i].wait_recv()
            obuf[pl.ds(b0 + i, 1)] = combine(send_buf[i], ypart[i])[None]
            oc = pltpu.make_async_copy(
                obuf.at[b0 + i], out_ref.at[b0 + i, 0], out_sems.at[i]
            )
            oc.start()
            out_copies.append(oc)

        ob0 = (1 - my_x) * NB
        for i in range(NB):
            rdmas[3 * i + 1].wait_recv()
            rdmas[3 * i + 2].wait_recv()
            obuf[pl.ds(ob0 + i, 1)] = combine(xpart[i], dpart[i])[None]
            oc = pltpu.make_async_copy(
                obuf.at[ob0 + i], out_ref.at[ob0 + i, 0], out_sems.at[NB + i]
            )
            oc.start()
            out_copies.append(oc)

        for r in rdmas:
            r.wait_send()
        for oc in out_copies:
            oc.wait()

    return pl.pallas_call(
        body,
        out_shape=jax.ShapeDtypeStruct((B, SQ, H, D), jnp.float32),
        in_specs=[
            pl.BlockSpec(memory_space=pl.ANY),
            pl.BlockSpec(memory_space=pl.ANY),
            pl.BlockSpec(memory_space=pl.ANY),
        ],
        input_output_aliases={0: 0},
        out_specs=pl.BlockSpec(memory_space=pl.ANY),
        scratch_shapes=[
            pltpu.VMEM((NB, H, D, SKV), jnp.float32),
            pltpu.VMEM((NB, H, D, SKV), jnp.float32),
            pltpu.VMEM((B, SQ, H, D), jnp.float32),
            pltpu.VMEM((NB, H, 3 * D), jnp.float32),
            pltpu.VMEM((NB, H, 3 * D), jnp.float32),
            pltpu.VMEM((NB, H, 3 * D), jnp.float32),
            pltpu.VMEM((NB, H, 3 * D), jnp.float32),
            pltpu.VMEM((B, H, D), jnp.float32),
            pltpu.SemaphoreType.DMA((2 * NB,)),
            pltpu.SemaphoreType.DMA((NB,)),
            pltpu.SemaphoreType.DMA((NB,)),
            pltpu.SemaphoreType.DMA((NB,)),
            pltpu.SemaphoreType.DMA((NB,)),
            pltpu.SemaphoreType.DMA((NB,)),
            pltpu.SemaphoreType.DMA((NB,)),
            pltpu.SemaphoreType.DMA((2 * NB,)),
            pltpu.SemaphoreType.DMA,
        ],
        compiler_params=pltpu.CompilerParams(
            collective_id=0,
            vmem_limit_bytes=96 * 1024 * 1024,
        ),
    )(Q, Kt, Vt)


# device time: 12436 ns/iter; 1.1824x vs baseline; 1.1824x over previous
import jax
import jax.numpy as jnp
from jax import lax
from jax.experimental import pallas as pl
from jax.experimental.pallas import tpu as pltpu

B, SQ, SKV, H, D = 8, 1, 512, 8, 64
NB = B // 2


def kernel(Q, K, V):
    scale = D ** -0.5
    Kt = jnp.transpose(K, (0, 2, 3, 1))
    Vt = jnp.transpose(V, (0, 2, 3, 1))

    def body(q_ref, k_ref, v_ref, out_ref, kbuf, vbuf,
             send_buf, ypart, xpart, dpart, obuf,
             sems, ysend_sems, yrecv_sems, xsend_sems, xrecv_sems,
             dsend_sems, drecv_sems, out_sems):
        my_x = lax.axis_index("x")
        my_y = lax.axis_index("y")
        my_z = lax.axis_index("z")
        nbr_y = (my_x, 1 - my_y, my_z)
        nbr_x = (1 - my_x, my_y, my_z)
        nbr_d = (1 - my_x, 1 - my_y, my_z)
        b0 = my_x * NB

        barrier_sem = pltpu.get_barrier_semaphore()
        for nbr in (nbr_y, nbr_x, nbr_d):
            pl.semaphore_signal(
                barrier_sem, inc=1, device_id=nbr,
                device_id_type=pl.DeviceIdType.MESH,
            )

        copies = []
        for i in range(NB):
            kc = pltpu.make_async_copy(
                k_ref.at[b0 + i], kbuf.at[i], sems.at[i]
            )
            vc = pltpu.make_async_copy(
                v_ref.at[b0 + i], vbuf.at[i], sems.at[NB + i]
            )
            kc.start()
            vc.start()
            copies.append((kc, vc))

        rdmas = []
        for i in range(NB):
            kc, vc = copies[i]
            q_b = q_ref[pl.ds(b0 + i, 1)].reshape(H, D)
            kc.wait()
            s = lax.dot_general(
                q_b, kbuf[i],
                (((1,), (1,)), ((0,), (0,))),
                preferred_element_type=jnp.float32,
            ) * scale
            m = jnp.max(s, axis=-1, keepdims=True)
            p = jnp.exp(s - m)
            l = jnp.sum(p, axis=-1, keepdims=True)
            vc.wait()
            o = lax.dot_general(
                p, vbuf[i],
                (((1,), (2,)), ((0,), (0,))),
                preferred_element_type=jnp.float32,
            )
            send_buf[i, :, 0:D] = o
            send_buf[i, :, D:2 * D] = jnp.broadcast_to(m, (H, D))
            send_buf[i, :, 2 * D:3 * D] = jnp.broadcast_to(l, (H, D))
            if i == 0:
                pl.semaphore_wait(barrier_sem, 3)
            for dst, ss, rs, nbr in (
                (ypart, ysend_sems, yrecv_sems, nbr_y),
                (xpart, xsend_sems, xrecv_sems, nbr_x),
                (dpart, dsend_sems, drecv_sems, nbr_d),
            ):
                rdma = pltpu.make_async_remote_copy(
                    src_ref=send_buf.at[i],
                    dst_ref=dst.at[i],
                    send_sem=ss.at[i],
                    recv_sem=rs.at[i],
                    device_id=nbr,
                    device_id_type=pl.DeviceIdType.MESH,
                )
                rdma.start()
                rdmas.append(rdma)

        def combine(mine, other):
            o1, m1, l1 = (mine[:, 0:D], mine[:, D:2 * D], mine[:, 2 * D:3 * D])
            o2, m2, l2 = (other[:, 0:D], other[:, D:2 * D], other[:, 2 * D:3 * D])
            mn = jnp.maximum(m1, m2)
            a1 = jnp.exp(m1 - mn)
            a2 = jnp.exp(m2 - mn)
            return (a1 * o1 + a2 * o2) / (a1 * l1 + a2 * l2)

        out_copies = []
        for i in range(NB):
            rdmas[3 * i].wait_recv()
            obuf[pl.ds(b0 + i, 1)] = combine(send_buf[i], ypart[i])[None]
            oc = pltpu.make_async_copy(
                obuf.at[b0 + i], out_ref.at[b0 + i, 0], out_sems.at[i]
            )
            oc.start()
            out_copies.append(oc)

        ob0 = (1 - my_x) * NB
        for i in range(NB):
            rdmas[3 * i + 1].wait_recv()
            rdmas[3 * i + 2].wait_recv()
            obuf[pl.ds(ob0 + i, 1)] = combine(xpart[i], dpart[i])[None]
            oc = pltpu.make_async_copy(
                obuf.at[ob0 + i], out_ref.at[ob0 + i, 0], out_sems.at[NB + i]
            )
            oc.start()
            out_copies.append(oc)

        for r in rdmas:
            r.wait_send()
        for oc in out_copies:
            oc.wait()

    return pl.pallas_call(
        body,
        out_shape=jax.ShapeDtypeStruct((B, SQ, H, D), jnp.float32),
        in_specs=[
            pl.BlockSpec(memory_space=pltpu.MemorySpace.VMEM),
            pl.BlockSpec(memory_space=pl.ANY),
            pl.BlockSpec(memory_space=pl.ANY),
        ],
        out_specs=pl.BlockSpec(memory_space=pl.ANY),
        scratch_shapes=[
            pltpu.VMEM((NB, H, D, SKV), jnp.float32),
            pltpu.VMEM((NB, H, D, SKV), jnp.float32),
            pltpu.VMEM((NB, H, 3 * D), jnp.float32),
            pltpu.VMEM((NB, H, 3 * D), jnp.float32),
            pltpu.VMEM((NB, H, 3 * D), jnp.float32),
            pltpu.VMEM((NB, H, 3 * D), jnp.float32),
            pltpu.VMEM((B, H, D), jnp.float32),
            pltpu.SemaphoreType.DMA((2 * NB,)),
            pltpu.SemaphoreType.DMA((NB,)),
            pltpu.SemaphoreType.DMA((NB,)),
            pltpu.SemaphoreType.DMA((NB,)),
            pltpu.SemaphoreType.DMA((NB,)),
            pltpu.SemaphoreType.DMA((NB,)),
            pltpu.SemaphoreType.DMA((NB,)),
            pltpu.SemaphoreType.DMA((2 * NB,)),
        ],
        compiler_params=pltpu.CompilerParams(
            collective_id=0,
            vmem_limit_bytes=96 * 1024 * 1024,
        ),
    )(Q, Kt, Vt)
